# Initial kernel scaffold; baseline (speedup 1.0000x reference)
import jax
import jax.numpy as jnp
from jax.experimental import pallas as pl


def kernel(entity_ids, entity_table):
    raise NotImplementedError("write your pallas kernel here")


# speedup vs baseline: 5.5187x; 5.5187x over previous
import functools

import jax
import jax.numpy as jnp
from jax import lax
from jax.experimental import pallas as pl
from jax.experimental.pallas import tpu as pltpu
from jax.experimental.pallas import tpu_sc as plsc

NC = 2
NS = 16
NW = NC * NS

V = 100
D = 3
TPAD = 320

CHUNK = 10240


def _sc_lookup(ids_flat, table_pad, n_flat):
    per_w = n_flat // NW
    n_chunks = per_w // CHUNK

    mesh = plsc.VectorSubcoreMesh(core_axis_name="c", subcore_axis_name="s")

    @functools.partial(
        pl.kernel,
        mesh=mesh,
        out_type=jax.ShapeDtypeStruct((n_flat * D,), jnp.float32),
        scratch_types=[
            pltpu.VMEM((TPAD,), jnp.float32),
            pltpu.VMEM((CHUNK,), jnp.int32),
            pltpu.VMEM((CHUNK * D,), jnp.float32),
        ],
        compiler_params=pltpu.CompilerParams(needs_layout_passes=False),
    )
    def k(table_hbm, ids_hbm, out_hbm, table_v, ids_v, out_v):
        wid = lax.axis_index("s") * NC + lax.axis_index("c")
        pltpu.sync_copy(table_hbm, table_v)
        base_w = wid * per_w

        iota16 = lax.iota(jnp.int32, 16)
        pos0 = iota16 * D

        def chunk_body(c, carry):
            base = base_w + c * CHUNK
            pltpu.sync_copy(ids_hbm.at[pl.ds(base, CHUNK)], ids_v)

            def grp(i, carry2):
                ids16 = ids_v[pl.ds(i * 16, 16)]
                addr = ids16 * D
                pos = pos0 + i * (16 * D)
                for d in range(D):
                    vals = plsc.load_gather(table_v, [addr + d])
                    plsc.store_scatter(out_v, [pos + d], vals)
                return carry2

            lax.fori_loop(0, CHUNK // 16, grp, 0, unroll=4)
            pltpu.sync_copy(out_v, out_hbm.at[pl.ds(base * D, CHUNK * D)])
            return carry

        lax.fori_loop(0, n_chunks, chunk_body, 0)

    return k(table_pad, ids_flat)


def kernel(entity_ids, entity_table):
    b0, b1 = entity_ids.shape
    n_flat = b0 * b1
    ids_flat = entity_ids.reshape(n_flat).astype(jnp.int32)
    table_pad = jnp.zeros((TPAD,), jnp.float32).at[: V * D].set(
        entity_table.reshape(V * D)
    )
    out_flat = _sc_lookup(ids_flat, table_pad, n_flat)
    return out_flat.reshape(b0, b1, D)


# speedup vs baseline: 75.7806x; 13.7315x over previous
import functools

import jax
import jax.numpy as jnp
from jax import lax
from jax.experimental import pallas as pl
from jax.experimental.pallas import tpu as pltpu
from jax.experimental.pallas import tpu_sc as plsc

NC = 2
NS = 16
NW = NC * NS

V = 100
D = 3
TPAD = 320


def _sc_lookup(ids_t, table_pad):
    t_dim, b_dim = ids_t.shape
    cols = b_dim // NW
    n_tr = t_dim // 8

    mesh = plsc.VectorSubcoreMesh(core_axis_name="c", subcore_axis_name="s")

    @functools.partial(
        pl.kernel,
        mesh=mesh,
        out_type=jax.ShapeDtypeStruct((D, t_dim, b_dim), jnp.float32),
        scratch_types=[
            pltpu.VMEM((TPAD,), jnp.float32),
            pltpu.VMEM((8, cols), jnp.int32),
            pltpu.VMEM((8, cols), jnp.float32),
            pltpu.VMEM((8, cols), jnp.float32),
            pltpu.VMEM((8, cols), jnp.float32),
        ],
        compiler_params=pltpu.CompilerParams(needs_layout_passes=False),
    )
    def k(table_hbm, ids_hbm, out_hbm, table_v, ids_v, o0, o1, o2):
        wid = lax.axis_index("s") * NC + lax.axis_index("c")
        pltpu.sync_copy(table_hbm, table_v)
        cbase = wid * cols
        outs = (o0, o1, o2)

        def tr_body(tr, carry):
            pltpu.sync_copy(
                ids_hbm.at[pl.ds(tr * 8, 8), pl.ds(cbase, cols)], ids_v
            )
            for r in range(8):

                def grp(g, carry2, r=r):
                    c = g * 16
                    ids16 = ids_v[r, pl.ds(c, 16)]
                    a = ids16 * D
                    for d in range(D):
                        outs[d][r, pl.ds(c, 16)] = plsc.load_gather(
                            table_v, [a + d]
                        )
                    return carry2

                lax.fori_loop(0, cols // 16, grp, 0, unroll=4)
            for d in range(D):
                pltpu.sync_copy(
                    outs[d], out_hbm.at[d, pl.ds(tr * 8, 8), pl.ds(cbase, cols)]
                )
            return carry

        lax.fori_loop(0, n_tr, tr_body, 0)

    return k(table_pad, ids_t)


def kernel(entity_ids, entity_table):
    ids_t = entity_ids.T.astype(jnp.int32)
    table_pad = jnp.zeros((TPAD,), jnp.float32).at[: V * D].set(
        entity_table.reshape(V * D)
    )
    out_t = _sc_lookup(ids_t, table_pad)
    return out_t.transpose(2, 1, 0)


# speedup vs baseline: 99.3026x; 1.3104x over previous
import functools

import jax
import jax.numpy as jnp
from jax import lax
from jax.experimental import pallas as pl
from jax.experimental.pallas import tpu as pltpu
from jax.experimental.pallas import tpu_sc as plsc

NC = 2
NS = 16
NW = NC * NS

V = 100
D = 3
TPAD = 320
SLAB = 256


def _sc_lookup(ids_t, table_pad):
    t_dim, b_dim = ids_t.shape
    wcols = b_dim // NW
    n_slabs = (t_dim // 8) * (wcols // SLAB)

    mesh = plsc.VectorSubcoreMesh(core_axis_name="c", subcore_axis_name="s")

    @functools.partial(
        pl.kernel,
        mesh=mesh,
        out_type=jax.ShapeDtypeStruct((D, t_dim, b_dim), jnp.float32),
        scratch_types=[
            pltpu.VMEM((TPAD,), jnp.float32),
            pltpu.VMEM((8, SLAB), jnp.int32),
            pltpu.VMEM((8, SLAB), jnp.int32),
            pltpu.VMEM((8, SLAB), jnp.float32),
            pltpu.VMEM((8, SLAB), jnp.float32),
            pltpu.VMEM((8, SLAB), jnp.float32),
            pltpu.VMEM((8, SLAB), jnp.float32),
            pltpu.VMEM((8, SLAB), jnp.float32),
            pltpu.VMEM((8, SLAB), jnp.float32),
            pltpu.SemaphoreType.DMA,
            pltpu.SemaphoreType.DMA,
            pltpu.SemaphoreType.DMA,
            pltpu.SemaphoreType.DMA,
        ],
        compiler_params=pltpu.CompilerParams(needs_layout_passes=False),
    )
    def k(table_hbm, ids_hbm, out_hbm, table_v,
          ids_a, ids_b, oa0, oa1, oa2, ob0, ob1, ob2,
          semi_a, semi_b, semo_a, semo_b):
        wid = lax.axis_index("s") * NC + lax.axis_index("c")
        pltpu.sync_copy(table_hbm, table_v)
        cbase = wid * wcols
        banks = (
            (ids_a, (oa0, oa1, oa2), semi_a, semo_a),
            (ids_b, (ob0, ob1, ob2), semi_b, semo_b),
        )

        def ids_src(s):
            tr = s // 2
            return ids_hbm.at[
                pl.ds(tr * 8, 8), pl.ds(cbase + (s % 2) * SLAB, SLAB)
            ]

        def out_dst(s, d):
            tr = s // 2
            return out_hbm.at[
                d, pl.ds(tr * 8, 8), pl.ds(cbase + (s % 2) * SLAB, SLAB)
            ]

        def start_ids(s, bank):
            pltpu.make_async_copy(ids_src(s), bank[0], bank[2]).start()

        def wait_ids(s, bank):
            pltpu.make_async_copy(ids_src(s), bank[0], bank[2]).wait()

        def start_outs(s, bank):
            for d in range(D):
                pltpu.make_async_copy(bank[1][d], out_dst(s, d), bank[3]).start()

        def wait_outs(s, bank):
            for d in range(D):
                pltpu.make_async_copy(bank[1][d], out_dst(s, d), bank[3]).wait()

        start_ids(0, banks[0])
        start_ids(1, banks[1])

        def body(j, carry):
            for b in (0, 1):
                bank = banks[b]
                s = 2 * j + b
                wait_ids(s, bank)

                @pl.when(j > 0)
                def _():
                    wait_outs(s - 2, bank)

                ids_v = bank[0]
                outs = bank[1]
                for r in range(8):

                    def grp(g, carry2, r=r):
                        c = g * 16
                        ids16 = ids_v[r, pl.ds(c, 16)]
                        a = ids16 * D
                        for d in range(D):
                            outs[d][r, pl.ds(c, 16)] = plsc.load_gather(
                                table_v, [a + d]
                            )
                        return carry2

                    lax.fori_loop(0, SLAB // 16, grp, 0, unroll=4)
                start_outs(s, bank)

                @pl.when(s + 2 < n_slabs)
                def _():
                    start_ids(s + 2, bank)

            return carry

        lax.fori_loop(0, n_slabs // 2, body, 0)
        wait_outs(n_slabs - 2, banks[0])
        wait_outs(n_slabs - 1, banks[1])

    return k(table_pad, ids_t)


def kernel(entity_ids, entity_table):
    ids_t = entity_ids.T.astype(jnp.int32)
    table_pad = jnp.zeros((TPAD,), jnp.float32).at[: V * D].set(
        entity_table.reshape(V * D)
    )
    out_t = _sc_lookup(ids_t, table_pad)
    return out_t.transpose(2, 1, 0)


# speedup vs baseline: 155.7407x; 1.5683x over previous
import functools

import jax
import jax.numpy as jnp
from jax import lax
from jax.experimental import pallas as pl
from jax.experimental.pallas import tpu as pltpu
from jax.experimental.pallas import tpu_sc as plsc

NC = 2
NS = 16
NW = NC * NS

V = 100
D = 3
VPAD = 128
SLAB = 256


def _sc_lookup(ids_t, table_cols):
    t_dim, b_dim = ids_t.shape
    wcols = b_dim // NW
    n_slabs = (t_dim // 8) * (wcols // SLAB)

    mesh = plsc.VectorSubcoreMesh(core_axis_name="c", subcore_axis_name="s")

    @functools.partial(
        pl.kernel,
        mesh=mesh,
        out_type=jax.ShapeDtypeStruct((D, t_dim, b_dim), jnp.float32),
        scratch_types=[
            pltpu.VMEM((VPAD,), jnp.float32),
            pltpu.VMEM((VPAD,), jnp.float32),
            pltpu.VMEM((VPAD,), jnp.float32),
            pltpu.VMEM((8, SLAB), jnp.int32),
            pltpu.VMEM((8, SLAB), jnp.int32),
            pltpu.VMEM((8, SLAB), jnp.float32),
            pltpu.VMEM((8, SLAB), jnp.float32),
            pltpu.VMEM((8, SLAB), jnp.float32),
            pltpu.VMEM((8, SLAB), jnp.float32),
            pltpu.VMEM((8, SLAB), jnp.float32),
            pltpu.VMEM((8, SLAB), jnp.float32),
            pltpu.SemaphoreType.DMA,
            pltpu.SemaphoreType.DMA,
            pltpu.SemaphoreType.DMA,
            pltpu.SemaphoreType.DMA,
        ],
        compiler_params=pltpu.CompilerParams(needs_layout_passes=False),
    )
    def k(tc0_hbm, tc1_hbm, tc2_hbm, ids_hbm, out_hbm, t0, t1, t2,
          ids_a, ids_b, oa0, oa1, oa2, ob0, ob1, ob2,
          semi_a, semi_b, semo_a, semo_b):
        wid = lax.axis_index("s") * NC + lax.axis_index("c")
        tcols = (t0, t1, t2)
        for d, src in enumerate((tc0_hbm, tc1_hbm, tc2_hbm)):
            pltpu.sync_copy(src, tcols[d])
        cbase = wid * wcols
        banks = (
            (ids_a, (oa0, oa1, oa2), semi_a, semo_a),
            (ids_b, (ob0, ob1, ob2), semi_b, semo_b),
        )

        def ids_src(s):
            tr = s // 2
            return ids_hbm.at[
                pl.ds(tr * 8, 8), pl.ds(cbase + (s % 2) * SLAB, SLAB)
            ]

        def out_dst(s, d):
            tr = s // 2
            return out_hbm.at[
                d, pl.ds(tr * 8, 8), pl.ds(cbase + (s % 2) * SLAB, SLAB)
            ]

        def start_ids(s, bank):
            pltpu.make_async_copy(ids_src(s), bank[0], bank[2]).start()

        def wait_ids(s, bank):
            pltpu.make_async_copy(ids_src(s), bank[0], bank[2]).wait()

        def start_outs(s, bank):
            for d in range(D):
                pltpu.make_async_copy(bank[1][d], out_dst(s, d), bank[3]).start()

        def wait_outs(s, bank):
            for d in range(D):
                pltpu.make_async_copy(bank[1][d], out_dst(s, d), bank[3]).wait()

        start_ids(0, banks[0])
        start_ids(1, banks[1])

        def body(j, carry):
            for b in (0, 1):
                bank = banks[b]
                s = 2 * j + b
                wait_ids(s, bank)

                @pl.when(j > 0)
                def _():
                    wait_outs(s - 2, bank)

                ids_v = bank[0]
                outs = bank[1]
                for r in range(8):

                    def grp4(q, carry2, r=r):
                        base = q * 64
                        idv = [
                            ids_v[r, pl.ds(base + 16 * kk, 16)] for kk in range(4)
                        ]
                        vals = [
                            [plsc.load_gather(tcols[d], [idv[kk]]) for d in range(D)]
                            for kk in range(4)
                        ]
                        for kk in range(4):
                            for d in range(D):
                                outs[d][r, pl.ds(base + 16 * kk, 16)] = vals[kk][d]
                        return carry2

                    lax.fori_loop(0, SLAB // 64, grp4, 0, unroll=2)
                start_outs(s, bank)

                @pl.when(s + 2 < n_slabs)
                def _():
                    start_ids(s + 2, bank)

            return carry

        lax.fori_loop(0, n_slabs // 2, body, 0)
        wait_outs(n_slabs - 2, banks[0])
        wait_outs(n_slabs - 1, banks[1])

    return k(table_cols[0], table_cols[1], table_cols[2], ids_t)


def kernel(entity_ids, entity_table):
    ids_t = entity_ids.T.astype(jnp.int32)
    tc = jnp.zeros((D, VPAD), jnp.float32).at[:, :V].set(entity_table.T)
    table_cols = (tc[0], tc[1], tc[2])
    out_t = _sc_lookup(ids_t, table_cols)
    return out_t.transpose(2, 1, 0)


# speedup vs baseline: 246.6769x; 1.5839x over previous
import functools

import jax
import jax.numpy as jnp
from jax import lax
from jax.experimental import pallas as pl
from jax.experimental.pallas import tpu as pltpu
from jax.experimental.pallas import tpu_sc as plsc

NC = 2
NS = 16
NW = NC * NS

V = 100
D = 3
VPAD = 128
SLAB = 512


def _sc_lookup(ids_t, table_cols):
    t_dim, b_dim = ids_t.shape
    wcols = b_dim // NW
    n_slabs = t_dim // 8

    mesh = plsc.VectorSubcoreMesh(core_axis_name="c", subcore_axis_name="s")

    @functools.partial(
        pl.kernel,
        mesh=mesh,
        out_type=jax.ShapeDtypeStruct((D, t_dim, b_dim), jnp.float32),
        scratch_types=[
            pltpu.VMEM((VPAD,), jnp.float32),
            pltpu.VMEM((VPAD,), jnp.float32),
            pltpu.VMEM((VPAD,), jnp.float32),
            pltpu.VMEM((8, SLAB), jnp.int32),
            pltpu.VMEM((8, SLAB), jnp.int32),
            pltpu.VMEM((8, SLAB), jnp.float32),
            pltpu.VMEM((8, SLAB), jnp.float32),
            pltpu.VMEM((8, SLAB), jnp.float32),
            pltpu.VMEM((8, SLAB), jnp.float32),
            pltpu.VMEM((8, SLAB), jnp.float32),
            pltpu.VMEM((8, SLAB), jnp.float32),
            pltpu.SemaphoreType.DMA,
            pltpu.SemaphoreType.DMA,
            pltpu.SemaphoreType.DMA,
            pltpu.SemaphoreType.DMA,
        ],
        compiler_params=pltpu.CompilerParams(needs_layout_passes=False),
    )
    def k(tc0_hbm, tc1_hbm, tc2_hbm, ids_hbm, out_hbm, t0, t1, t2,
          ids_a, ids_b, oa0, oa1, oa2, ob0, ob1, ob2,
          semi_a, semi_b, semo_a, semo_b):
        wid = lax.axis_index("s") * NC + lax.axis_index("c")
        tcols = (t0, t1, t2)
        for d, src in enumerate((tc0_hbm, tc1_hbm, tc2_hbm)):
            pltpu.sync_copy(src, tcols[d])
        cbase = wid * wcols
        banks = (
            (ids_a, (oa0, oa1, oa2), semi_a, semo_a),
            (ids_b, (ob0, ob1, ob2), semi_b, semo_b),
        )

        def ids_src(s):
            return ids_hbm.at[pl.ds(s * 8, 8), pl.ds(cbase, SLAB)]

        def out_dst(s, d):
            return out_hbm.at[d, pl.ds(s * 8, 8), pl.ds(cbase, SLAB)]

        def start_ids(s, bank):
            pltpu.make_async_copy(ids_src(s), bank[0], bank[2]).start()

        def wait_ids(s, bank):
            pltpu.make_async_copy(ids_src(s), bank[0], bank[2]).wait()

        def start_outs(s, bank):
            for d in range(D):
                pltpu.make_async_copy(bank[1][d], out_dst(s, d), bank[3]).start()

        def wait_outs(s, bank):
            for d in range(D):
                pltpu.make_async_copy(bank[1][d], out_dst(s, d), bank[3]).wait()

        def compute(bank):
            ids_v = bank[0]
            outs = bank[1]
            for r in range(8):

                @plsc.parallel_loop(0, SLAB // 16, unroll=4)
                def _(g, r=r):
                    c = g * 16
                    ids16 = ids_v[r, pl.ds(c, 16)]
                    for d in range(D):
                        outs[d][r, pl.ds(c, 16)] = plsc.load_gather(
                            tcols[d], [ids16]
                        )

        def process(s, bank):
            wait_ids(s, bank)

            @pl.when(s >= 2)
            def _():
                wait_outs(s - 2, bank)

            compute(bank)
            start_outs(s, bank)

            @pl.when(s + 2 < n_slabs)
            def _():
                start_ids(s + 2, bank)

        start_ids(0, banks[0])
        start_ids(1, banks[1])

        def body(j, carry):
            process(2 * j, banks[0])
            process(2 * j + 1, banks[1])
            return carry

        lax.fori_loop(0, n_slabs // 2, body, 0)
        process(n_slabs - 1, banks[0])
        wait_outs(n_slabs - 2, banks[1])
        wait_outs(n_slabs - 1, banks[0])

    return k(table_cols[0], table_cols[1], table_cols[2], ids_t)


def kernel(entity_ids, entity_table):
    ids_t = entity_ids.T.astype(jnp.int32)
    tc = jnp.zeros((D, VPAD), jnp.float32).at[:, :V].set(entity_table.T)
    table_cols = (tc[0], tc[1], tc[2])
    out_t = _sc_lookup(ids_t, table_cols)
    return out_t.transpose(2, 1, 0)
